# baseline (speedup 1.0000x reference)
import functools

import jax
import jax.numpy as jnp
from jax import lax
from jax.experimental import pallas as pl
from jax.experimental.pallas import tpu as pltpu
from jax.experimental.pallas import tpu_sc as plsc

EMBEDDING_DIM = 64
PACKED_DIM = 2 * EMBEDDING_DIM
NUM_CORES = 2
NUM_SUBCORES = 16
NUM_WORKERS = NUM_CORES * NUM_SUBCORES
NBUF = 2
CHUNK = 800
VB = 4096


def _row_major_table(wt):
    dim, vocab = wt.shape

    def body(wt_ref, o_ref):
        t3 = wt_ref[...].T.reshape(VB // 2, 2, dim)
        o_ref[...] = jnp.concatenate([t3[:, 0, :], t3[:, 1, :]], axis=1)

    return pl.pallas_call(
        body,
        grid=(pl.cdiv(vocab, VB),),
        in_specs=[pl.BlockSpec((dim, VB), lambda i: (0, i))],
        out_specs=pl.BlockSpec((VB // 2, PACKED_DIM), lambda i: (i, 0)),
        out_shape=jax.ShapeDtypeStruct((vocab // 2, PACKED_DIM), jnp.float32),
        compiler_params=pltpu.CompilerParams(dimension_semantics=("parallel",)),
    )(wt)


def _sc_gather(table, idx):
    n = idx.shape[0]
    per_worker = n // NUM_WORKERS
    n_chunks = per_worker // CHUNK
    mesh = plsc.VectorSubcoreMesh(core_axis_name="c", subcore_axis_name="s")

    @functools.partial(
        pl.kernel,
        mesh=mesh,
        compiler_params=pltpu.CompilerParams(use_tc_tiling_on_sc=False),
        out_type=jax.ShapeDtypeStruct((n, EMBEDDING_DIM), jnp.float32),
        scratch_types=[
            pltpu.VMEM((per_worker,), jnp.int32),
        ]
        + [pltpu.VMEM((CHUNK, EMBEDDING_DIM), jnp.float32) for _ in range(NBUF)]
        + [pltpu.SemaphoreType.DMA for _ in range(2 * NBUF)],
    )
    def gather_k(table_hbm, idx_hbm, out_hbm, idx_v, *scratch):
        bufs = scratch[:NBUF]
        gsems = scratch[NBUF : 2 * NBUF]
        wsems = scratch[2 * NBUF :]
        wid = lax.axis_index("s") * NUM_CORES + lax.axis_index("c")
        base = wid * per_worker
        pltpu.sync_copy(idx_hbm.at[pl.ds(base, per_worker)], idx_v)

        def start_gather(c):
            b = c % NBUF
            return pltpu.async_copy(
                table_hbm.at[idx_v.at[pl.ds(c * CHUNK, CHUNK)]], bufs[b], gsems[b]
            )

        gh = [None] * NBUF
        wr = [None] * NBUF
        for c in range(NBUF - 1):
            gh[c % NBUF] = start_gather(c)
        for c in range(n_chunks):
            b = c % NBUF
            nxt = c + NBUF - 1
            if nxt < n_chunks:
                nb = nxt % NBUF
                if wr[nb] is not None:
                    wr[nb].wait()
                gh[nb] = start_gather(nxt)
            gh[b].wait()
            wr[b] = pltpu.async_copy(
                bufs[b], out_hbm.at[pl.ds(base + c * CHUNK, CHUNK)], wsems[b]
            )
        for w in wr:
            if w is not None:
                w.wait()

    return gather_k(table, idx)


def _to_batch_minor(packed, seq, batch):
    half = batch // 2

    def body(in_ref, o_ref):
        r3 = in_ref[0].reshape(half, 2, EMBEDDING_DIM)
        o_ref[0] = jnp.concatenate([r3[:, 0, :].T, r3[:, 1, :].T], axis=1)

    return pl.pallas_call(
        body,
        grid=(seq,),
        in_specs=[pl.BlockSpec((1, half, PACKED_DIM), lambda s: (s, 0, 0))],
        out_specs=pl.BlockSpec((1, EMBEDDING_DIM, batch), lambda s: (s, 0, 0)),
        out_shape=jax.ShapeDtypeStruct((seq, EMBEDDING_DIM, batch), jnp.float32),
        compiler_params=pltpu.CompilerParams(dimension_semantics=("parallel",)),
    )(packed)


def kernel(x, weight):
    batch, seq = x.shape
    n = batch * seq
    half = batch // 2
    idx = jnp.transpose(x.reshape(2, half, seq), (2, 1, 0)).reshape(n)
    packed_table = _row_major_table(weight.T)
    table = packed_table.reshape(weight.shape[0], EMBEDDING_DIM)
    rows = _sc_gather(table, idx)
    packed = rows.reshape(seq, half, PACKED_DIM)
    p = _to_batch_minor(packed, seq, batch)
    return jnp.transpose(p, (2, 0, 1))


# speedup vs baseline: 1.4842x; 1.4842x over previous
import functools

import jax
import jax.numpy as jnp
from jax import lax
from jax.experimental import pallas as pl
from jax.experimental.pallas import tpu as pltpu
from jax.experimental.pallas import tpu_sc as plsc

EMBEDDING_DIM = 64
PACKED_DIM = 2 * EMBEDDING_DIM
NUM_CORES = 2
NUM_SUBCORES = 16
NUM_WORKERS = NUM_CORES * NUM_SUBCORES
NSLOT = 5
CHUNK = 256
VB = 4096


def _row_major_table(wt):
    dim, vocab = wt.shape

    def body(wt_ref, o_ref):
        t3 = wt_ref[...].T.reshape(VB // 2, 2, dim)
        o_ref[...] = jnp.concatenate([t3[:, 0, :], t3[:, 1, :]], axis=1)

    return pl.pallas_call(
        body,
        grid=(pl.cdiv(vocab, VB),),
        in_specs=[pl.BlockSpec((dim, VB), lambda i: (0, i))],
        out_specs=pl.BlockSpec((VB // 2, PACKED_DIM), lambda i: (i, 0)),
        out_shape=jax.ShapeDtypeStruct((vocab // 2, PACKED_DIM), jnp.float32),
        compiler_params=pltpu.CompilerParams(dimension_semantics=("parallel",)),
    )(wt)


def _sc_gather(table, idx, batch):
    n = idx.shape[0]
    half = batch // 2
    per_worker = n // NUM_WORKERS
    n_chunks = per_worker // CHUNK
    n_groups = n_chunks // NSLOT
    mesh = plsc.VectorSubcoreMesh(core_axis_name="c", subcore_axis_name="s")

    @functools.partial(
        pl.kernel,
        mesh=mesh,
        compiler_params=pltpu.CompilerParams(use_tc_tiling_on_sc=False),
        out_type=jax.ShapeDtypeStruct((n // 2, 2, EMBEDDING_DIM), jnp.float32),
        scratch_types=[
            pltpu.VMEM((per_worker,), jnp.int32),
        ]
        + [pltpu.VMEM((CHUNK, EMBEDDING_DIM), jnp.float32) for _ in range(NSLOT)]
        + [pltpu.SemaphoreType.DMA for _ in range(2 * NSLOT)],
    )
    def gather_k(table_hbm, idx_hbm, out_hbm, idx_v, *scratch):
        bufs = scratch[:NSLOT]
        gsems = scratch[NSLOT : 2 * NSLOT]
        wsems = scratch[2 * NSLOT :]
        wid = lax.axis_index("s") * NUM_CORES + lax.axis_index("c")
        base = wid * per_worker
        pltpu.sync_copy(idx_hbm.at[pl.ds(base, per_worker)], idx_v)

        def start_gather(c, s):
            return pltpu.async_copy(
                table_hbm.at[idx_v.at[pl.ds(c * CHUNK, CHUNK)]], bufs[s], gsems[s]
            )

        def start_write(c, s):
            o = base + c * CHUNK
            sq = o // batch
            rem = o % batch
            p = rem // half
            j0 = rem % half
            t0 = sq * half + j0
            return pltpu.async_copy(
                bufs[s], out_hbm.at[pl.ds(t0, CHUNK), p], wsems[s]
            )

        for s in range(NSLOT):
            start_gather(s, s)

        @pl.loop(0, n_groups)
        def _(g):
            L = g * NSLOT
            handles = []
            for s in range(NSLOT):
                pltpu.make_async_copy(
                    table_hbm.at[idx_v.at[pl.ds(0, CHUNK)]], bufs[s], gsems[s]
                ).wait()
                handles.append(start_write(L + s, s))
            for s in range(NSLOT):
                handles[s].wait()

                @pl.when(g < n_groups - 1)
                def _():
                    start_gather(L + s + NSLOT, s)

    return gather_k(table, idx)


def _to_batch_minor(packed, seq, batch):
    half = batch // 2

    def body(in_ref, o_ref):
        v = in_ref[0]
        o_ref[0] = jnp.concatenate(
            [v[:, :EMBEDDING_DIM].T, v[:, EMBEDDING_DIM:].T], axis=1
        )

    return pl.pallas_call(
        body,
        grid=(seq,),
        in_specs=[pl.BlockSpec((1, half, PACKED_DIM), lambda s: (s, 0, 0))],
        out_specs=pl.BlockSpec((1, EMBEDDING_DIM, batch), lambda s: (s, 0, 0)),
        out_shape=jax.ShapeDtypeStruct((seq, EMBEDDING_DIM, batch), jnp.float32),
        compiler_params=pltpu.CompilerParams(dimension_semantics=("parallel",)),
    )(packed)


def kernel(x, weight):
    batch, seq = x.shape
    n = batch * seq
    half = batch // 2
    idx = x.T.reshape(n)
    packed_table = _row_major_table(weight.T)
    table = packed_table.reshape(weight.shape[0], EMBEDDING_DIM)
    out3 = _sc_gather(table, idx, batch)
    packed = out3.reshape(seq, half, PACKED_DIM)
    p = _to_batch_minor(packed, seq, batch)
    return jnp.transpose(p, (2, 0, 1))
